# baseline (speedup 1.0000x reference)
import functools

import jax
import jax.numpy as jnp
from jax import lax
from jax.experimental import pallas as pl
from jax.experimental.pallas import tpu as pltpu
from jax.experimental.pallas import tpu_sc as plsc

NUM_CORES = 2
NUM_SUBCORES = 16
NW = NUM_CORES * NUM_SUBCORES
CHUNK = 128
NBUF = 8


def _mesh():
    return plsc.VectorSubcoreMesh(
        core_axis_name="c", subcore_axis_name="s",
        num_cores=NUM_CORES, num_subcores=NUM_SUBCORES)


def _make_deg_kernel(NP, C):
    rows_per_tile = NP // NUM_SUBCORES
    K0 = ((C // 2 + 7) // 8) * 8
    K1 = C - K0

    @functools.partial(
        pl.kernel,
        out_type=jax.ShapeDtypeStruct((NUM_CORES * NP,), jnp.float32),
        mesh=_mesh(),
        compiler_params=pltpu.CompilerParams(use_tc_tiling_on_sc=False),
        scratch_types=[
            pltpu.VMEM((max(K0, K1), CHUNK), jnp.int32),
            pltpu.VMEM((CHUNK,), jnp.float32),
            pltpu.VMEM((rows_per_tile,), jnp.float32),
            pltpu.VMEM_SHARED((NP,), jnp.float32),
            pltpu.SemaphoreType.DMA,
        ],
    )
    def deg_kernel(dst_hbm, out_hbm, dstv, ones, stage, accum, sem):
        c = lax.axis_index("c")
        s = lax.axis_index("s")

        zeros16 = jnp.zeros((16,), jnp.float32)
        ones16 = jnp.ones((16,), jnp.float32)

        def zero_body(i, _):
            stage[pl.ds(i * 16, 16)] = zeros16
            return 0
        lax.fori_loop(0, rows_per_tile // 16, zero_body, 0)

        def ones_body(i, _):
            ones[pl.ds(i * 16, 16)] = ones16
            return 0
        lax.fori_loop(0, CHUNK // 16, ones_body, 0)

        pltpu.sync_copy(stage, accum.at[pl.ds(s * rows_per_tile, rows_per_tile)])
        plsc.subcore_barrier()

        def count_edges(Kc, base):
            pltpu.sync_copy(dst_hbm.at[pl.ds(base, Kc)],
                            dstv.at[pl.ds(0, Kc)])

            def edge_body(j, _):
                pltpu.sync_copy(ones, accum.at[dstv.at[j]], add=True)
                return 0
            lax.fori_loop(0, Kc, edge_body, 0)

        @pl.when(c == 0)
        def _():
            count_edges(K0, s * C)

        if K1 > 0:
            @pl.when(c == 1)
            def _():
                count_edges(K1, s * C + K0)

        plsc.subcore_barrier()
        pltpu.sync_copy(accum.at[pl.ds(s * rows_per_tile, rows_per_tile)], stage)
        pltpu.sync_copy(stage, out_hbm.at[pl.ds(c * NP + s * rows_per_tile,
                                                rows_per_tile)])

    return deg_kernel


def _make_scatter_kernel(NP, K0, K1, H):
    rows_per_tile = NP // NUM_SUBCORES

    @functools.partial(
        pl.kernel,
        out_type=jax.ShapeDtypeStruct((NUM_CORES * NP, H), jnp.float32),
        mesh=_mesh(),
        compiler_params=pltpu.CompilerParams(use_tc_tiling_on_sc=False),
        scratch_types=[
            pltpu.VMEM((max(K0, K1), CHUNK), jnp.int32),
            pltpu.VMEM((max(K0, K1), CHUNK), jnp.int32),
            [pltpu.VMEM((CHUNK, H), jnp.float32) for _ in range(NBUF)],
            pltpu.VMEM((rows_per_tile, H), jnp.float32),
            pltpu.VMEM_SHARED((NP, H), jnp.float32),
            pltpu.SemaphoreType.DMA,
            [pltpu.SemaphoreType.DMA for _ in range(NBUF)],
        ],
    )
    def scatter_kernel(hp_hbm, hpb_hbm, src_hbm, dst_hbm, out_hbm,
                       srcv, dstv, rows, stage, accum, gsem, ssem):
        c = lax.axis_index("c")
        s = lax.axis_index("s")

        zeros16 = jnp.zeros((16,), jnp.float32)

        def zero_body(i, _):
            def zcol(t, _):
                stage[i, pl.ds(t * 16, 16)] = zeros16
                return 0
            lax.fori_loop(0, H // 16, zcol, 0)
            return 0
        lax.fori_loop(0, rows_per_tile, zero_body, 0)

        pltpu.sync_copy(stage, accum.at[pl.ds(s * rows_per_tile, rows_per_tile)])
        plsc.subcore_barrier()

        def start_gather(src_ref, j, b):
            pltpu.async_copy(src_ref.at[srcv.at[j]], rows[b], gsem)

        def wait_gather(b):
            pltpu.make_async_copy(hp_hbm.at[pl.ds(0, CHUNK)], rows[b],
                                  gsem).wait()

        def start_scatter(j, b):
            pltpu.async_copy(rows[b], accum.at[dstv.at[j]], ssem[b], add=True)

        def drain_scatter(b):
            pltpu.make_async_copy(hp_hbm.at[pl.ds(0, CHUNK)], rows[b],
                                  ssem[b]).wait()

        Kc = jnp.where(c == 0, K0, K1)
        base = jnp.where(c == 0, s * K0, NUM_SUBCORES * K0 + s * K1)
        pltpu.sync_copy(src_hbm.at[pl.ds(base, K0)], srcv)
        pltpu.sync_copy(dst_hbm.at[pl.ds(base, K0)], dstv)

        for b in range(NBUF):
            start_gather(hp_hbm, b, b)
        for b in range(NBUF):
            wait_gather(b)
            start_scatter(b, b)

        def edge_body(i, _):
            for b in range(NBUF):
                drain_scatter(b)
                start_gather(hp_hbm, i * NBUF + b, b)
            for b in range(NBUF):
                wait_gather(b)
                start_scatter(i * NBUF + b, b)
            return 0
        lax.fori_loop(1, Kc // NBUF, edge_body, 0)
        for b in range(NBUF):
            drain_scatter(b)

        plsc.subcore_barrier()
        pltpu.sync_copy(accum.at[pl.ds(s * rows_per_tile, rows_per_tile)], stage)
        pltpu.sync_copy(stage, out_hbm.at[pl.ds(c * NP + s * rows_per_tile,
                                                rows_per_tile)])

    return scatter_kernel


def _dinv_call(deg2, NP):
    def body(deg_ref, o_ref):
        d = deg_ref[pl.ds(0, NP)] + deg_ref[pl.ds(NP, NP)] + 1.0
        o_ref[...] = 1.0 / jnp.sqrt(d)
    return pl.pallas_call(
        body, out_shape=jax.ShapeDtypeStruct((NP,), jnp.float32))(deg2)


def _mm0_call(x, W0, dinv, BN):
    N, D = x.shape
    H = W0.shape[1]
    grid = N // BN

    def body(x_ref, w_ref, dv_ref, o_ref):
        h = jnp.dot(x_ref[...], w_ref[...], preferred_element_type=jnp.float32)
        o_ref[...] = h * dv_ref[...]

    return pl.pallas_call(
        body,
        grid=(grid,),
        in_specs=[
            pl.BlockSpec((BN, D), lambda i: (i, 0)),
            pl.BlockSpec((D, H), lambda i: (0, 0)),
            pl.BlockSpec((BN, 1), lambda i: (i, 0)),
        ],
        out_specs=pl.BlockSpec((BN, H), lambda i: (i, 0)),
        out_shape=jax.ShapeDtypeStruct((N, H), jnp.float32),
    )(x, W0, dinv)


def _step_call(s3, hp, dinv, b, W, BN):
    N, H = hp.shape
    NP = s3.shape[1]
    grid = N // BN

    def body(s0_ref, s1_ref, hp_ref, dv_ref, b_ref, w_ref, o_ref):
        dv = dv_ref[...]
        y = dv * (s0_ref[0] + s1_ref[0] + hp_ref[...]) + b_ref[...]
        y = jnp.maximum(y, 0.0)
        h = jnp.dot(y, w_ref[...], preferred_element_type=jnp.float32)
        o_ref[...] = h * dv

    return pl.pallas_call(
        body,
        grid=(grid,),
        in_specs=[
            pl.BlockSpec((1, BN, H), lambda i: (0, i, 0)),
            pl.BlockSpec((1, BN, H), lambda i: (1, i, 0)),
            pl.BlockSpec((BN, H), lambda i: (i, 0)),
            pl.BlockSpec((BN, 1), lambda i: (i, 0)),
            pl.BlockSpec((H,), lambda i: (0,)),
            pl.BlockSpec((H, H), lambda i: (0, 0)),
        ],
        out_specs=pl.BlockSpec((BN, H), lambda i: (i, 0)),
        out_shape=jax.ShapeDtypeStruct((N, H), jnp.float32),
    )(s3, s3, hp, dinv, b, W)


def _head_call(s3, hp, dinv, b2, fW1, fb1, fW2, fb2, fW3, fb3, BN):
    N, H = hp.shape
    C = fW3.shape[1]
    grid = N // BN

    def body(s0_ref, s1_ref, hp_ref, dv_ref, b_ref,
             w1_ref, b1_ref, w2_ref, b2_ref, w3_ref, b3_ref, o_ref):
        dv = dv_ref[...]
        y = dv * (s0_ref[0] + s1_ref[0] + hp_ref[...]) + b_ref[...]
        y = jnp.maximum(y, 0.0)
        y = jnp.maximum(
            jnp.dot(y, w1_ref[...], preferred_element_type=jnp.float32)
            + b1_ref[...], 0.0)
        y = jnp.maximum(
            jnp.dot(y, w2_ref[...], preferred_element_type=jnp.float32)
            + b2_ref[...], 0.0)
        y = jnp.dot(y, w3_ref[...], preferred_element_type=jnp.float32) + b3_ref[...]
        m = jnp.max(y, axis=1, keepdims=True)
        e = jnp.exp(y - m)
        o_ref[...] = (y - m) - jnp.log(jnp.sum(e, axis=1, keepdims=True))

    return pl.pallas_call(
        body,
        grid=(grid,),
        in_specs=[
            pl.BlockSpec((1, BN, H), lambda i: (0, i, 0)),
            pl.BlockSpec((1, BN, H), lambda i: (1, i, 0)),
            pl.BlockSpec((BN, H), lambda i: (i, 0)),
            pl.BlockSpec((BN, 1), lambda i: (i, 0)),
            pl.BlockSpec((H,), lambda i: (0,)),
            pl.BlockSpec((H, H), lambda i: (0, 0)),
            pl.BlockSpec((H,), lambda i: (0,)),
            pl.BlockSpec((H, H), lambda i: (0, 0)),
            pl.BlockSpec((H,), lambda i: (0,)),
            pl.BlockSpec((H, C), lambda i: (0, 0)),
            pl.BlockSpec((C,), lambda i: (0,)),
        ],
        out_specs=pl.BlockSpec((BN, C), lambda i: (i, 0)),
        out_shape=jax.ShapeDtypeStruct((N, C), jnp.float32),
    )(s3, s3, hp, dinv, b2, fW1, fb1, fW2, fb2, fW3, fb3)


def kernel(x, edge_index, W0, b0, W1, b1, W2, b2, fW1, fb1, fW2, fb2, fW3, fb3):
    N, D = x.shape
    H = W0.shape[1]
    E = edge_index.shape[1]

    NP = ((N + 1 + 127) // 128) * 128
    C = -(-E // (CHUNK * NUM_SUBCORES))
    C = ((C + 39) // 40) * 40
    K0 = (C * 4) // 5
    K1 = C - K0
    EP = (NUM_SUBCORES * C + (K0 - K1)) * CHUNK
    BN = 2000 if N % 2000 == 0 else 8 * (N // 8)

    pad = EP - E
    src = jnp.concatenate(
        [edge_index[0], jnp.zeros((pad,), jnp.int32)]).reshape(-1, CHUNK)
    dst = jnp.concatenate(
        [edge_index[1], jnp.full((pad,), N, jnp.int32)]).reshape(-1, CHUNK)

    deg2 = _make_deg_kernel(NP, C)(dst)
    dinv = _dinv_call(deg2, NP).reshape(NP, 1)

    scatter = _make_scatter_kernel(NP, K0, K1, H)

    hp = _mm0_call(x, W0, dinv, BN)
    s3 = scatter(hp, hp, src, dst).reshape(2, NP, H)
    hp = _step_call(s3, hp, dinv, b0, W1, BN)
    s3 = scatter(hp, hp, src, dst).reshape(2, NP, H)
    hp = _step_call(s3, hp, dinv, b1, W2, BN)
    s3 = scatter(hp, hp, src, dst).reshape(2, NP, H)
    return _head_call(s3, hp, dinv, b2, fW1, fb1, fW2, fb2, fW3, fb3, BN)


# speedup vs baseline: 1.0308x; 1.0308x over previous
import functools

import jax
import jax.numpy as jnp
from jax import lax
from jax.experimental import pallas as pl
from jax.experimental.pallas import tpu as pltpu
from jax.experimental.pallas import tpu_sc as plsc

NUM_CORES = 2
NUM_SUBCORES = 16
NW = NUM_CORES * NUM_SUBCORES
CHUNK = 128
NBUF = 8


def _mesh():
    return plsc.VectorSubcoreMesh(
        core_axis_name="c", subcore_axis_name="s",
        num_cores=NUM_CORES, num_subcores=NUM_SUBCORES)


def _make_deg_kernel(NP, C):
    rows_per_tile = NP // NUM_SUBCORES
    K0 = ((C // 2 + 7) // 8) * 8
    K1 = C - K0

    @functools.partial(
        pl.kernel,
        out_type=jax.ShapeDtypeStruct((NUM_CORES * NP,), jnp.float32),
        mesh=_mesh(),
        compiler_params=pltpu.CompilerParams(use_tc_tiling_on_sc=False),
        scratch_types=[
            pltpu.VMEM((max(K0, K1), CHUNK), jnp.int32),
            pltpu.VMEM((CHUNK,), jnp.float32),
            pltpu.VMEM((rows_per_tile,), jnp.float32),
            pltpu.VMEM_SHARED((NP,), jnp.float32),
            pltpu.SemaphoreType.DMA,
        ],
    )
    def deg_kernel(dst_hbm, out_hbm, dstv, ones, stage, accum, sem):
        c = lax.axis_index("c")
        s = lax.axis_index("s")

        zeros16 = jnp.zeros((16,), jnp.float32)
        ones16 = jnp.ones((16,), jnp.float32)

        def zero_body(i, _):
            stage[pl.ds(i * 16, 16)] = zeros16
            return 0
        lax.fori_loop(0, rows_per_tile // 16, zero_body, 0)

        def ones_body(i, _):
            ones[pl.ds(i * 16, 16)] = ones16
            return 0
        lax.fori_loop(0, CHUNK // 16, ones_body, 0)

        pltpu.sync_copy(stage, accum.at[pl.ds(s * rows_per_tile, rows_per_tile)])
        plsc.subcore_barrier()

        def count_edges(Kc, base):
            pltpu.sync_copy(dst_hbm.at[pl.ds(base, Kc)],
                            dstv.at[pl.ds(0, Kc)])

            def edge_body(j, _):
                pltpu.sync_copy(ones, accum.at[dstv.at[j]], add=True)
                return 0
            lax.fori_loop(0, Kc, edge_body, 0)

        @pl.when(c == 0)
        def _():
            count_edges(K0, s * C)

        if K1 > 0:
            @pl.when(c == 1)
            def _():
                count_edges(K1, s * C + K0)

        plsc.subcore_barrier()
        pltpu.sync_copy(accum.at[pl.ds(s * rows_per_tile, rows_per_tile)], stage)
        pltpu.sync_copy(stage, out_hbm.at[pl.ds(c * NP + s * rows_per_tile,
                                                rows_per_tile)])

    return deg_kernel


def _make_scatter_kernel(NP, K0, K1, H):
    rows_per_tile = NP // NUM_SUBCORES

    @functools.partial(
        pl.kernel,
        out_type=jax.ShapeDtypeStruct((NUM_CORES * NP, H), jnp.float32),
        mesh=_mesh(),
        compiler_params=pltpu.CompilerParams(use_tc_tiling_on_sc=False),
        scratch_types=[
            pltpu.VMEM((max(K0, K1), CHUNK), jnp.int32),
            pltpu.VMEM((max(K0, K1), CHUNK), jnp.int32),
            [pltpu.VMEM((CHUNK, H), jnp.float32) for _ in range(NBUF)],
            pltpu.VMEM((rows_per_tile, H), jnp.float32),
            pltpu.VMEM_SHARED((NP, H), jnp.float32),
            pltpu.SemaphoreType.DMA,
            [pltpu.SemaphoreType.DMA for _ in range(NBUF)],
        ],
    )
    def scatter_kernel(hp_hbm, src_hbm, dst_hbm, out_hbm,
                       srcv, dstv, rows, stage, accum, gsem, ssem):
        c = lax.axis_index("c")
        s = lax.axis_index("s")

        zeros16 = jnp.zeros((16,), jnp.float32)

        def zero_body(i, _):
            def zcol(t, _):
                stage[i, pl.ds(t * 16, 16)] = zeros16
                return 0
            lax.fori_loop(0, H // 16, zcol, 0)
            return 0
        lax.fori_loop(0, rows_per_tile, zero_body, 0)

        pltpu.sync_copy(stage, accum.at[pl.ds(s * rows_per_tile, rows_per_tile)])
        plsc.subcore_barrier()

        def start_gather(src_ref, j, b):
            pltpu.async_copy(src_ref.at[srcv.at[j]], rows[b], gsem)

        def wait_gather(b):
            pltpu.make_async_copy(hp_hbm.at[pl.ds(0, CHUNK)], rows[b],
                                  gsem).wait()

        def start_scatter(j, b):
            pltpu.async_copy(rows[b], accum.at[dstv.at[j]], ssem[b], add=True)

        def drain_scatter(b):
            pltpu.make_async_copy(hp_hbm.at[pl.ds(0, CHUNK)], rows[b],
                                  ssem[b]).wait()

        def process_edges(Kc, base):
            pltpu.sync_copy(src_hbm.at[pl.ds(base, Kc)],
                            srcv.at[pl.ds(0, Kc)])
            pltpu.sync_copy(dst_hbm.at[pl.ds(base, Kc)],
                            dstv.at[pl.ds(0, Kc)])
            for b in range(NBUF):
                start_gather(hp_hbm, b, b)
            for b in range(NBUF):
                wait_gather(b)
                start_scatter(b, b)

            def edge_body(i, _):
                for b in range(NBUF):
                    drain_scatter(b)
                    start_gather(hp_hbm, i * NBUF + b, b)
                for b in range(NBUF):
                    wait_gather(b)
                    start_scatter(i * NBUF + b, b)
                return 0
            lax.fori_loop(1, Kc // NBUF, edge_body, 0)
            for b in range(NBUF):
                drain_scatter(b)

        @pl.when(c == 0)
        def _():
            process_edges(K0, s * K0)

        if K1 > 0:
            @pl.when(c == 1)
            def _():
                process_edges(K1, NUM_SUBCORES * K0 + s * K1)

        plsc.subcore_barrier()
        pltpu.sync_copy(accum.at[pl.ds(s * rows_per_tile, rows_per_tile)], stage)
        pltpu.sync_copy(stage, out_hbm.at[pl.ds(c * NP + s * rows_per_tile,
                                                rows_per_tile)])

    return scatter_kernel


def _dinv_call(deg2, NP):
    def body(deg_ref, o_ref):
        d = deg_ref[pl.ds(0, NP)] + deg_ref[pl.ds(NP, NP)] + 1.0
        o_ref[...] = 1.0 / jnp.sqrt(d)
    return pl.pallas_call(
        body, out_shape=jax.ShapeDtypeStruct((NP,), jnp.float32))(deg2)


def _mm0_call(x, W0, dinv, BN):
    N, D = x.shape
    H = W0.shape[1]
    grid = N // BN

    def body(x_ref, w_ref, dv_ref, o_ref):
        h = jnp.dot(x_ref[...], w_ref[...], preferred_element_type=jnp.float32)
        o_ref[...] = h * dv_ref[...]

    return pl.pallas_call(
        body,
        grid=(grid,),
        in_specs=[
            pl.BlockSpec((BN, D), lambda i: (i, 0)),
            pl.BlockSpec((D, H), lambda i: (0, 0)),
            pl.BlockSpec((BN, 1), lambda i: (i, 0)),
        ],
        out_specs=pl.BlockSpec((BN, H), lambda i: (i, 0)),
        out_shape=jax.ShapeDtypeStruct((N, H), jnp.float32),
    )(x, W0, dinv)


def _step_call(s3, hp, dinv, b, W, BN):
    N, H = hp.shape
    NP = s3.shape[1]
    grid = N // BN

    def body(s0_ref, s1_ref, hp_ref, dv_ref, b_ref, w_ref, o_ref):
        dv = dv_ref[...]
        y = dv * (s0_ref[0] + s1_ref[0] + hp_ref[...]) + b_ref[...]
        y = jnp.maximum(y, 0.0)
        h = jnp.dot(y, w_ref[...], preferred_element_type=jnp.float32)
        o_ref[...] = h * dv

    return pl.pallas_call(
        body,
        grid=(grid,),
        in_specs=[
            pl.BlockSpec((1, BN, H), lambda i: (0, i, 0)),
            pl.BlockSpec((1, BN, H), lambda i: (1, i, 0)),
            pl.BlockSpec((BN, H), lambda i: (i, 0)),
            pl.BlockSpec((BN, 1), lambda i: (i, 0)),
            pl.BlockSpec((H,), lambda i: (0,)),
            pl.BlockSpec((H, H), lambda i: (0, 0)),
        ],
        out_specs=pl.BlockSpec((BN, H), lambda i: (i, 0)),
        out_shape=jax.ShapeDtypeStruct((N, H), jnp.float32),
    )(s3, s3, hp, dinv, b, W)


def _head_call(s3, hp, dinv, b2, fW1, fb1, fW2, fb2, fW3, fb3, BN):
    N, H = hp.shape
    C = fW3.shape[1]
    grid = N // BN

    def body(s0_ref, s1_ref, hp_ref, dv_ref, b_ref,
             w1_ref, b1_ref, w2_ref, b2_ref, w3_ref, b3_ref, o_ref):
        dv = dv_ref[...]
        y = dv * (s0_ref[0] + s1_ref[0] + hp_ref[...]) + b_ref[...]
        y = jnp.maximum(y, 0.0)
        y = jnp.maximum(
            jnp.dot(y, w1_ref[...], preferred_element_type=jnp.float32)
            + b1_ref[...], 0.0)
        y = jnp.maximum(
            jnp.dot(y, w2_ref[...], preferred_element_type=jnp.float32)
            + b2_ref[...], 0.0)
        y = jnp.dot(y, w3_ref[...], preferred_element_type=jnp.float32) + b3_ref[...]
        m = jnp.max(y, axis=1, keepdims=True)
        e = jnp.exp(y - m)
        o_ref[...] = (y - m) - jnp.log(jnp.sum(e, axis=1, keepdims=True))

    return pl.pallas_call(
        body,
        grid=(grid,),
        in_specs=[
            pl.BlockSpec((1, BN, H), lambda i: (0, i, 0)),
            pl.BlockSpec((1, BN, H), lambda i: (1, i, 0)),
            pl.BlockSpec((BN, H), lambda i: (i, 0)),
            pl.BlockSpec((BN, 1), lambda i: (i, 0)),
            pl.BlockSpec((H,), lambda i: (0,)),
            pl.BlockSpec((H, H), lambda i: (0, 0)),
            pl.BlockSpec((H,), lambda i: (0,)),
            pl.BlockSpec((H, H), lambda i: (0, 0)),
            pl.BlockSpec((H,), lambda i: (0,)),
            pl.BlockSpec((H, C), lambda i: (0, 0)),
            pl.BlockSpec((C,), lambda i: (0,)),
        ],
        out_specs=pl.BlockSpec((BN, C), lambda i: (i, 0)),
        out_shape=jax.ShapeDtypeStruct((N, C), jnp.float32),
    )(s3, s3, hp, dinv, b2, fW1, fb1, fW2, fb2, fW3, fb3)


def kernel(x, edge_index, W0, b0, W1, b1, W2, b2, fW1, fb1, fW2, fb2, fW3, fb3):
    N, D = x.shape
    H = W0.shape[1]
    E = edge_index.shape[1]

    NP = ((N + 1 + 127) // 128) * 128
    C = -(-E // (CHUNK * NUM_SUBCORES))
    C = ((C + 39) // 40) * 40
    K0 = (C * 4) // 5
    K1 = C - K0
    EP = NUM_SUBCORES * C * CHUNK
    BN = 2000 if N % 2000 == 0 else 8 * (N // 8)

    pad = EP - E
    src = jnp.concatenate(
        [edge_index[0], jnp.zeros((pad,), jnp.int32)]).reshape(-1, CHUNK)
    dst = jnp.concatenate(
        [edge_index[1], jnp.full((pad,), N, jnp.int32)]).reshape(-1, CHUNK)

    deg2 = _make_deg_kernel(NP, C)(dst)
    dinv = _dinv_call(deg2, NP).reshape(NP, 1)

    scatter = _make_scatter_kernel(NP, K0, K1, H)

    hp = _mm0_call(x, W0, dinv, BN)
    s3 = scatter(hp, src, dst).reshape(2, NP, H)
    hp = _step_call(s3, hp, dinv, b0, W1, BN)
    s3 = scatter(hp, src, dst).reshape(2, NP, H)
    hp = _step_call(s3, hp, dinv, b1, W2, BN)
    s3 = scatter(hp, src, dst).reshape(2, NP, H)
    return _head_call(s3, hp, dinv, b2, fW1, fb1, fW2, fb2, fW3, fb3, BN)
